# baseline (speedup 1.0000x reference)
import functools

import jax
import jax.numpy as jnp
from jax import lax
from jax.experimental import pallas as pl
from jax.experimental.pallas import tpu as pltpu
from jax.experimental.pallas import tpu_sc as plsc

_B = 4096
_F = 26
_C = 1000
_NC = 2
_NS = 16
_NW = _NC * _NS
_WIN = 8
_NWIN = _C // _WIN
_NCHUNK = _F * _NWIN
_L = 16
_PER_W = (_NCHUNK + _NW - 1) // _NW


def _sc_onehot(idx_hbm, off_hbm, on_hbm, depth_hbm, out_hbm,
               idx_v, off_v, on_v, depth_v, buf):
    w = lax.axis_index("s") * _NC + lax.axis_index("c")

    pltpu.sync_copy(off_hbm, off_v)
    pltpu.sync_copy(on_hbm, on_v)
    pltpu.sync_copy(depth_hbm, depth_v)
    off = off_v[...]
    on = on_v[...]
    depth = depth_v[...]
    lane = lax.iota(jnp.int32, _L)

    def gen(win):
        c0 = win * _WIN

        def seg(g, _):
            iv = idx_v[pl.ds(g * _L, _L)]
            ivm = jnp.where(iv < depth, iv, -1)
            for r in range(_WIN):
                buf[r, pl.ds(g * _L, _L)] = jnp.where(ivm == c0 + r, on, off)
            return _
        lax.fori_loop(0, _B // _L, seg, None)

    def step(t, _):
        cid = w + _NW * t

        @pl.when(cid < _NCHUNK)
        def _do():
            f = cid // _NWIN
            win = cid % _NWIN
            pltpu.sync_copy(idx_hbm.at[f], idx_v)
            gen(win)
            pltpu.sync_copy(buf, out_hbm.at[f, pl.ds(win * _WIN, _WIN), :])
        return _

    lax.fori_loop(0, _PER_W, step, None)


def kernel(indices, depth, values):
    idx_t = indices.T
    off16 = jnp.full((_L,), values[0], dtype=jnp.float32)
    on16 = jnp.full((_L,), values[1], dtype=jnp.float32)
    depth16 = jnp.full((_L,), depth, dtype=jnp.int32)

    mesh = plsc.VectorSubcoreMesh(core_axis_name="c", subcore_axis_name="s")
    k = functools.partial(
        pl.kernel,
        mesh=mesh,
        out_type=jax.ShapeDtypeStruct((_F, _C, _B), jnp.float32),
        scratch_types=[
            pltpu.VMEM((_B,), jnp.int32),
            pltpu.VMEM((_L,), jnp.float32),
            pltpu.VMEM((_L,), jnp.float32),
            pltpu.VMEM((_L,), jnp.int32),
            pltpu.VMEM((_WIN, _B), jnp.float32),
        ],
    )(_sc_onehot)
    out_t = k(idx_t, off16, on16, depth16)
    return out_t.transpose(2, 0, 1)


# speedup vs baseline: 4.3280x; 4.3280x over previous
import jax
import jax.numpy as jnp
from jax.experimental import pallas as pl
from jax.experimental.pallas import tpu as pltpu

_B = 4096
_F = 26
_C = 1000
_CB = 1000


def _onehot_block(idx_ref, depth_ref, values_ref, out_ref):
    c0 = pl.program_id(1) * _CB
    idx = idx_ref[pl.ds(pl.program_id(0), 1), :]
    depth = depth_ref[0]
    idx_eff = jnp.where(idx < depth, idx, -1)
    cls = jax.lax.broadcasted_iota(jnp.int32, (_CB, _B), 0) + c0
    out_ref[...] = jnp.where(cls == idx_eff, values_ref[1], values_ref[0])


def kernel(indices, depth, values):
    depth_arr = jnp.asarray(depth, dtype=jnp.int32).reshape(1)
    idx_t = indices.T
    out_t = pl.pallas_call(
        _onehot_block,
        grid=(_F, _C // _CB),
        in_specs=[
            pl.BlockSpec((_F, _B), lambda f, c: (0, 0)),
            pl.BlockSpec(memory_space=pltpu.SMEM),
            pl.BlockSpec(memory_space=pltpu.SMEM),
        ],
        out_specs=pl.BlockSpec((None, _CB, _B), lambda f, c: (f, c, 0)),
        out_shape=jax.ShapeDtypeStruct((_F, _C, _B), jnp.float32),
    )(idx_t, depth_arr, values)
    return out_t.transpose(2, 0, 1)
